# baseline (speedup 1.0000x reference)
import jax
import jax.numpy as jnp
from jax import lax
from jax.experimental import pallas as pl
from jax.experimental.pallas import tpu as pltpu
from jax.experimental.pallas import tpu_sc as plsc

B, S, F = 4, 8192, 768
K = 16
LANES = 16
N_CORES = 2
N_SUBCORES = 16
N_WORKERS = N_CORES * N_SUBCORES
F_PER_W = F // (N_WORKERS // B)
G_PER_W = F_PER_W // LANES
CHUNK = 512
N_CHUNKS = S // CHUNK
NBUF = 2
ROWS_PER_STEP = 8
SUBBLOCKS = 8
ROWS_PER_ITER = ROWS_PER_STEP * SUBBLOCKS
ITERS = CHUNK // ROWS_PER_ITER

NEG_INF = float("-inf")


_SORT8 = ((0, 1), (2, 3), (4, 5), (6, 7), (0, 2), (1, 3), (1, 2),
          (4, 6), (5, 7), (5, 6), (0, 4), (1, 5), (2, 6), (3, 7),
          (2, 4), (3, 5), (1, 2), (3, 4), (5, 6))


def _merge8(R, vs):
    s = list(vs)
    for (i, j) in _SORT8:
        hi = jnp.maximum(s[i], s[j])
        lo = jnp.minimum(s[i], s[j])
        s[i], s[j] = hi, lo
    C = list(R)
    for i in range(8, K):
        C[i] = jnp.maximum(R[i], s[15 - i])
    for d in (8, 4, 2, 1):
        for i in range(K):
            if i % (2 * d) < d:
                hi = jnp.maximum(C[i], C[i + d])
                lo = jnp.minimum(C[i], C[i + d])
                C[i], C[i + d] = hi, lo
    return tuple(C)


def _maxtree8(vs):
    w01 = jnp.maximum(vs[0], vs[1])
    w23 = jnp.maximum(vs[2], vs[3])
    w45 = jnp.maximum(vs[4], vs[5])
    w67 = jnp.maximum(vs[6], vs[7])
    return jnp.maximum(jnp.maximum(w01, w23), jnp.maximum(w45, w67))


def _topk_body(x_hbm, out_hbm, buf, rbuf, obuf, sem):
    cid = lax.axis_index("c")
    sid = lax.axis_index("s")
    wid = sid * N_CORES + cid
    b = wid // (N_WORKERS // B)
    f0 = (wid % (N_WORKERS // B)) * F_PER_W

    def start_dma(c):
        k = lax.rem(c, NBUF)
        pltpu.async_copy(
            x_hbm.at[b, pl.ds(c * CHUNK, CHUNK), pl.ds(f0, F_PER_W)],
            buf.at[k], sem.at[k])

    def wait_dma(c):
        k = lax.rem(c, NBUF)
        pltpu.make_async_copy(
            x_hbm.at[0, pl.ds(0, CHUNK), pl.ds(0, F_PER_W)],
            buf.at[k], sem.at[k]).wait()

    start_dma(0)
    start_dma(1)

    for g in range(G_PER_W):
        def init_j(j, _, g=g):
            rbuf[g, j, :] = jnp.full((LANES,), NEG_INF, jnp.float32)
            return 0
        lax.fori_loop(0, K, init_j, 0)

    def process(kk):
        for g in range(G_PER_W):
            col0 = g * LANES
            R = tuple(rbuf[g, j, :] for j in range(K))

            def step(i, R, col0=col0, kk=kk):
                r15_pre = R[K - 1]
                vss = []
                cnts = []
                for u in range(SUBBLOCKS):
                    base = i * ROWS_PER_ITER + u * ROWS_PER_STEP
                    vs = [buf[kk, base + r, col0:col0 + LANES]
                          for r in range(ROWS_PER_STEP)]
                    vss.append(vs)
                    w = _maxtree8(vs)
                    cnt = plsc.all_reduce_population_count(w > r15_pre)
                    cnts.append(cnt[0])
                total = cnts[0]
                for u in range(1, SUBBLOCKS):
                    total = total + cnts[u]

                def merge_block(R):
                    for u in range(SUBBLOCKS):
                        R = lax.cond(cnts[u] > 0,
                                     lambda R, u=u: _merge8(R, vss[u]),
                                     lambda R: R, R)
                    return R

                return lax.cond(total > 0, merge_block, lambda R: R, R)

            R = lax.fori_loop(0, ITERS, step, R)
            for j in range(K):
                rbuf[g, j, :] = R[j]

    def outer(c, _):
        wait_dma(c)
        process(lax.rem(c, NBUF))

        @pl.when(c + NBUF < N_CHUNKS)
        def _():
            start_dma(c + NBUF)
        return 0

    lax.fori_loop(0, N_CHUNKS, outer, 0)

    for g in range(G_PER_W):
        def out_j(j, _, g=g):
            obuf[j, g * LANES:(g + 1) * LANES] = rbuf[g, j, :]
            return 0
        lax.fori_loop(0, K, out_j, 0)

    pltpu.sync_copy(obuf, out_hbm.at[b, :, pl.ds(f0, F_PER_W)])


@jax.jit
def _kmax_sc(x):
    mesh = plsc.VectorSubcoreMesh(core_axis_name="c", subcore_axis_name="s")
    f = pl.kernel(
        _topk_body,
        mesh=mesh,
        compiler_params=pltpu.CompilerParams(
            use_tc_tiling_on_sc=False, needs_layout_passes=False),
        out_type=jax.ShapeDtypeStruct((B, K, F), jnp.float32),
        scratch_types=[
            pltpu.VMEM((NBUF, CHUNK, F_PER_W), jnp.float32),
            pltpu.VMEM((G_PER_W, K, LANES), jnp.float32),
            pltpu.VMEM((K, F_PER_W), jnp.float32),
            pltpu.SemaphoreType.DMA((NBUF,)),
        ],
    )
    return f(x)


def kernel(x):
    return _kmax_sc(x)


# speedup vs baseline: 1.0817x; 1.0817x over previous
import jax
import jax.numpy as jnp
from jax import lax
from jax.experimental import pallas as pl
from jax.experimental.pallas import tpu as pltpu
from jax.experimental.pallas import tpu_sc as plsc

B, S, F = 4, 8192, 768
K = 16
LANES = 16
N_CORES = 2
N_SUBCORES = 16
N_WORKERS = N_CORES * N_SUBCORES
F_PER_W = F // (N_WORKERS // B)
G_PER_W = F_PER_W // LANES
CHUNK = 512
N_CHUNKS = S // CHUNK
NBUF = 2
ROWS_PER_STEP = 8
SUBBLOCKS = 4
ROWS_PER_ITER = ROWS_PER_STEP * SUBBLOCKS
ITERS = CHUNK // ROWS_PER_ITER

NEG_INF = float("-inf")


_SORT8 = ((0, 1), (2, 3), (4, 5), (6, 7), (0, 2), (1, 3), (1, 2),
          (4, 6), (5, 7), (5, 6), (0, 4), (1, 5), (2, 6), (3, 7),
          (2, 4), (3, 5), (1, 2), (3, 4), (5, 6))


def _merge8(R, vs):
    s = list(vs)
    for (i, j) in _SORT8:
        hi = jnp.maximum(s[i], s[j])
        lo = jnp.minimum(s[i], s[j])
        s[i], s[j] = hi, lo
    C = list(R)
    for i in range(8, K):
        C[i] = jnp.maximum(R[i], s[15 - i])
    for d in (8, 4, 2, 1):
        for i in range(K):
            if i % (2 * d) < d:
                hi = jnp.maximum(C[i], C[i + d])
                lo = jnp.minimum(C[i], C[i + d])
                C[i], C[i + d] = hi, lo
    return tuple(C)


def _maxtree8(vs):
    w01 = jnp.maximum(vs[0], vs[1])
    w23 = jnp.maximum(vs[2], vs[3])
    w45 = jnp.maximum(vs[4], vs[5])
    w67 = jnp.maximum(vs[6], vs[7])
    return jnp.maximum(jnp.maximum(w01, w23), jnp.maximum(w45, w67))


def _topk_body(x_hbm, out_hbm, buf, rbuf, obuf, sem):
    cid = lax.axis_index("c")
    sid = lax.axis_index("s")
    wid = sid * N_CORES + cid
    b = wid // (N_WORKERS // B)
    f0 = (wid % (N_WORKERS // B)) * F_PER_W

    def start_dma(c):
        k = lax.rem(c, NBUF)
        pltpu.async_copy(
            x_hbm.at[b, pl.ds(c * CHUNK, CHUNK), pl.ds(f0, F_PER_W)],
            buf.at[k], sem.at[k])

    def wait_dma(c):
        k = lax.rem(c, NBUF)
        pltpu.make_async_copy(
            x_hbm.at[0, pl.ds(0, CHUNK), pl.ds(0, F_PER_W)],
            buf.at[k], sem.at[k]).wait()

    start_dma(0)
    start_dma(1)

    for g in range(G_PER_W):
        def init_j(j, _, g=g):
            rbuf[g, j, :] = jnp.full((LANES,), NEG_INF, jnp.float32)
            return 0
        lax.fori_loop(0, K, init_j, 0)

    def process(kk):
        for g in range(G_PER_W):
            col0 = g * LANES
            R = tuple(rbuf[g, j, :] for j in range(K))

            def step(i, R, col0=col0, kk=kk):
                r15_pre = R[K - 1]
                vss = []
                cnts = []
                for u in range(SUBBLOCKS):
                    base = i * ROWS_PER_ITER + u * ROWS_PER_STEP
                    vs = [buf[kk, base + r, col0:col0 + LANES]
                          for r in range(ROWS_PER_STEP)]
                    vss.append(vs)
                    w = _maxtree8(vs)
                    cnt = plsc.all_reduce_population_count(w > r15_pre)
                    cnts.append(cnt[0])
                total = cnts[0]
                for u in range(1, SUBBLOCKS):
                    total = total + cnts[u]

                def merge_block(R):
                    for u in range(SUBBLOCKS):
                        R = lax.cond(cnts[u] > 0,
                                     lambda R, u=u: _merge8(R, vss[u]),
                                     lambda R: R, R)
                    return R

                return lax.cond(total > 0, merge_block, lambda R: R, R)

            R = lax.fori_loop(0, ITERS, step, R)
            for j in range(K):
                rbuf[g, j, :] = R[j]

    def outer(c, _):
        wait_dma(c)
        process(lax.rem(c, NBUF))

        @pl.when(c + NBUF < N_CHUNKS)
        def _():
            start_dma(c + NBUF)
        return 0

    lax.fori_loop(0, N_CHUNKS, outer, 0)

    for g in range(G_PER_W):
        def out_j(j, _, g=g):
            obuf[j, g * LANES:(g + 1) * LANES] = rbuf[g, j, :]
            return 0
        lax.fori_loop(0, K, out_j, 0)

    pltpu.sync_copy(obuf, out_hbm.at[b, :, pl.ds(f0, F_PER_W)])


@jax.jit
def _kmax_sc(x):
    mesh = plsc.VectorSubcoreMesh(core_axis_name="c", subcore_axis_name="s")
    f = pl.kernel(
        _topk_body,
        mesh=mesh,
        compiler_params=pltpu.CompilerParams(
            use_tc_tiling_on_sc=False, needs_layout_passes=False),
        out_type=jax.ShapeDtypeStruct((B, K, F), jnp.float32),
        scratch_types=[
            pltpu.VMEM((NBUF, CHUNK, F_PER_W), jnp.float32),
            pltpu.VMEM((G_PER_W, K, LANES), jnp.float32),
            pltpu.VMEM((K, F_PER_W), jnp.float32),
            pltpu.SemaphoreType.DMA((NBUF,)),
        ],
    )
    return f(x)


def kernel(x):
    return _kmax_sc(x)


# speedup vs baseline: 1.1338x; 1.0482x over previous
import jax
import jax.numpy as jnp
from jax import lax
from jax.experimental import pallas as pl
from jax.experimental.pallas import tpu as pltpu
from jax.experimental.pallas import tpu_sc as plsc

B, S, F = 4, 8192, 768
K = 16
LANES = 16
N_CORES = 2
N_SUBCORES = 16
N_WORKERS = N_CORES * N_SUBCORES
F_PER_W = F // (N_WORKERS // B)
G_PER_W = F_PER_W // LANES
CHUNK = 512
N_CHUNKS = S // CHUNK
NBUF = 2
ROWS_PER_STEP = 8
SUBBLOCKS = 4
ROWS_PER_ITER = ROWS_PER_STEP * SUBBLOCKS
ITERS = CHUNK // ROWS_PER_ITER

NEG_INF = float("-inf")


_SORT8 = ((0, 1), (2, 3), (4, 5), (6, 7), (0, 2), (1, 3), (1, 2),
          (4, 6), (5, 7), (5, 6), (0, 4), (1, 5), (2, 6), (3, 7),
          (2, 4), (3, 5), (1, 2), (3, 4), (5, 6))


def _merge8(R, vs):
    s = list(vs)
    for (i, j) in _SORT8:
        hi = jnp.maximum(s[i], s[j])
        lo = jnp.minimum(s[i], s[j])
        s[i], s[j] = hi, lo
    C = list(R)
    for i in range(8, K):
        C[i] = jnp.maximum(R[i], s[15 - i])
    for d in (8, 4, 2, 1):
        for i in range(K):
            if i % (2 * d) < d:
                hi = jnp.maximum(C[i], C[i + d])
                lo = jnp.minimum(C[i], C[i + d])
                C[i], C[i + d] = hi, lo
    return tuple(C)


def _maxtree8(vs):
    w01 = jnp.maximum(vs[0], vs[1])
    w23 = jnp.maximum(vs[2], vs[3])
    w45 = jnp.maximum(vs[4], vs[5])
    w67 = jnp.maximum(vs[6], vs[7])
    return jnp.maximum(jnp.maximum(w01, w23), jnp.maximum(w45, w67))


def _topk_body(x_hbm, out_hbm, buf, rbuf, obuf, sem):
    cid = lax.axis_index("c")
    sid = lax.axis_index("s")
    wid = sid * N_CORES + cid
    b = wid // (N_WORKERS // B)
    f0 = (wid % (N_WORKERS // B)) * F_PER_W

    def start_dma(c):
        k = lax.rem(c, NBUF)
        pltpu.async_copy(
            x_hbm.at[b, pl.ds(c * CHUNK, CHUNK), pl.ds(f0, F_PER_W)],
            buf.at[k], sem.at[k])

    def wait_dma(c):
        k = lax.rem(c, NBUF)
        pltpu.make_async_copy(
            x_hbm.at[0, pl.ds(0, CHUNK), pl.ds(0, F_PER_W)],
            buf.at[k], sem.at[k]).wait()

    start_dma(0)
    start_dma(1)

    for g in range(G_PER_W):
        def init_j(j, _, g=g):
            rbuf[g, j, :] = jnp.full((LANES,), NEG_INF, jnp.float32)
            return 0
        lax.fori_loop(0, K, init_j, 0)

    def process(kk):
        for g in range(G_PER_W):
            col0 = g * LANES
            R = tuple(rbuf[g, j, :] for j in range(K))

            def step(i, R, col0=col0, kk=kk):
                r15_pre = R[K - 1]
                vss = []
                cnts = []
                for u in range(SUBBLOCKS):
                    base = i * ROWS_PER_ITER + u * ROWS_PER_STEP
                    vs = [buf[kk, base + r, col0:col0 + LANES]
                          for r in range(ROWS_PER_STEP)]
                    vss.append(vs)
                    w = _maxtree8(vs)
                    cnt = plsc.all_reduce_population_count(w > r15_pre)
                    cnts.append(cnt[0])
                for u in range(SUBBLOCKS):
                    R = lax.cond(cnts[u] > 0,
                                 lambda R, u=u: _merge8(R, vss[u]),
                                 lambda R: R, R)
                return R

            R = lax.fori_loop(0, ITERS, step, R)
            for j in range(K):
                rbuf[g, j, :] = R[j]

    def outer(c, _):
        wait_dma(c)
        process(lax.rem(c, NBUF))

        @pl.when(c + NBUF < N_CHUNKS)
        def _():
            start_dma(c + NBUF)
        return 0

    lax.fori_loop(0, N_CHUNKS, outer, 0)

    for g in range(G_PER_W):
        def out_j(j, _, g=g):
            obuf[j, g * LANES:(g + 1) * LANES] = rbuf[g, j, :]
            return 0
        lax.fori_loop(0, K, out_j, 0)

    pltpu.sync_copy(obuf, out_hbm.at[b, :, pl.ds(f0, F_PER_W)])


@jax.jit
def _kmax_sc(x):
    mesh = plsc.VectorSubcoreMesh(core_axis_name="c", subcore_axis_name="s")
    f = pl.kernel(
        _topk_body,
        mesh=mesh,
        compiler_params=pltpu.CompilerParams(
            use_tc_tiling_on_sc=False, needs_layout_passes=False),
        out_type=jax.ShapeDtypeStruct((B, K, F), jnp.float32),
        scratch_types=[
            pltpu.VMEM((NBUF, CHUNK, F_PER_W), jnp.float32),
            pltpu.VMEM((G_PER_W, K, LANES), jnp.float32),
            pltpu.VMEM((K, F_PER_W), jnp.float32),
            pltpu.SemaphoreType.DMA((NBUF,)),
        ],
    )
    return f(x)


def kernel(x):
    return _kmax_sc(x)
